# Initial kernel scaffold; baseline (speedup 1.0000x reference)
import jax
import jax.numpy as jnp
from jax.experimental import pallas as pl


def kernel(x, conv1_w, bn1_g, bn1_b, L1_b0_conv1, L1_b0_bn1_g, L1_b0_bn1_b, L1_b0_conv2, L1_b0_bn2_g, L1_b0_bn2_b, L1_b1_conv1, L1_b1_bn1_g, L1_b1_bn1_b, L1_b1_conv2, L1_b1_bn2_g, L1_b1_bn2_b, L2_b0_conv1, L2_b0_bn1_g, L2_b0_bn1_b, L2_b0_conv2, L2_b0_bn2_g, L2_b0_bn2_b, L2_b0_down_conv, L2_b0_down_bn_g, L2_b0_down_bn_b, L2_b1_conv1, L2_b1_bn1_g, L2_b1_bn1_b, L2_b1_conv2, L2_b1_bn2_g, L2_b1_bn2_b, L3_b0_conv1, L3_b0_bn1_g, L3_b0_bn1_b, L3_b0_conv2, L3_b0_bn2_g, L3_b0_bn2_b, L3_b0_down_conv, L3_b0_down_bn_g, L3_b0_down_bn_b, L3_b1_conv1, L3_b1_bn1_g, L3_b1_bn1_b, L3_b1_conv2, L3_b1_bn2_g, L3_b1_bn2_b, L4_b0_conv1, L4_b0_bn1_g, L4_b0_bn1_b, L4_b0_conv2, L4_b0_bn2_g, L4_b0_bn2_b, L4_b0_down_conv, L4_b0_down_bn_g, L4_b0_down_bn_b, L4_b1_conv1, L4_b1_bn1_g, L4_b1_bn1_b, L4_b1_conv2, L4_b1_bn2_g, L4_b1_bn2_b, fc_w, fc_b):
    raise NotImplementedError("write your pallas kernel here")


# speedup vs baseline: 3.2623x; 3.2623x over previous
import jax
import jax.numpy as jnp
from jax import lax
from jax.experimental import pallas as pl
from jax.experimental.pallas import tpu as pltpu

_VMEM_LIMIT = 48 * 1024 * 1024
_EPS = 1e-5
_BF16 = jnp.bfloat16
_F32 = jnp.float32


def _rup8(v):
    return -(-v // 8) * 8


def _bn_coeffs(s, g, b, inv_cnt):
    mean = s[0:1, :] * inv_cnt
    var = jnp.maximum(s[1:2, :] * inv_cnt - mean * mean, 0.0)
    scale = g * lax.rsqrt(var + _EPS)
    shift = b - mean * scale
    return scale, shift


def _zero_pad_to_scratch(scr, y, H, W, P, R):
    c = y.shape[-1]
    cl = scr.shape[-1]
    dt = scr.dtype
    scr[0:1, :, :] = jnp.zeros((1, P, cl), dt)
    scr[H + 1:R, :, :] = jnp.zeros((R - H - 1, P, cl), dt)
    scr[1:H + 1, 0:1, :] = jnp.zeros((H, 1, cl), dt)
    scr[1:H + 1, W + 1:P, :] = jnp.zeros((H, P - W - 1, cl), dt)
    scr[1:H + 1, 1:W + 1, 0:c] = y.astype(dt)


def _taps_matmul(flat, w_ref, k, P, M, cin, concat):
    if concat:
        av = jnp.concatenate(
            [flat[di * P + dj:di * P + dj + M, :]
             for di in range(k) for dj in range(k)], axis=1)
        return jnp.dot(av, w_ref[...], preferred_element_type=_F32)
    acc = None
    for di in range(k):
        for dj in range(k):
            d = jnp.dot(flat[di * P + dj:di * P + dj + M, :],
                        w_ref[di * k + dj], preferred_element_type=_F32)
            acc = d if acc is None else acc + d
    return acc


def _store_stats(so_ref, acc, P=None, W=None):
    if P is not None and W < P:
        col = lax.broadcasted_iota(jnp.int32, (acc.shape[0], 1), 0) % P
        acc = jnp.where(col < W, acc, 0.0)
    so_ref[0, 0:1, :] = jnp.sum(acc, axis=0, keepdims=True)
    so_ref[0, 1:2, :] = jnp.sum(acc * acc, axis=0, keepdims=True)


def _full_spec(shape):
    nd = len(shape)
    return pl.BlockSpec(shape, lambda i: (0,) * nd)


def _call(body, n, in_specs, args, out_shapes, out_specs, scratch=()):
    return pl.pallas_call(
        body,
        out_shape=out_shapes,
        grid=(n,),
        in_specs=in_specs,
        out_specs=out_specs,
        scratch_shapes=list(scratch),
        compiler_params=pltpu.CompilerParams(
            dimension_semantics=("parallel",),
            vmem_limit_bytes=_VMEM_LIMIT),
    )(*args)


def _make_s1_body(k, H, W, cin, P, R, M, concat, Pin, inv_cnt):
    prologue = Pin is not None

    def body(*refs):
        if prologue:
            x_ref, s_ref, g_ref, b_ref, w_ref, o_ref, so_ref, scr = refs
            scale, shift = _bn_coeffs(s_ref[...], g_ref[...], b_ref[...],
                                      inv_cnt)
            raw = x_ref[0].reshape(H, Pin, cin)[:, :W, :]
            y = jnp.maximum(raw.astype(_F32) * scale[None] + shift[None],
                            0.0).astype(_BF16)
        else:
            x_ref, w_ref, o_ref, so_ref, scr = refs
            y = x_ref[0]
        _zero_pad_to_scratch(scr, y, H, W, P, R)
        flat = scr[...].reshape(R * P, cin)
        acc = _taps_matmul(flat, w_ref, k, P, M, cin, concat)
        _store_stats(so_ref, acc, P, W)
        o_ref[0] = acc

    return body


def _conv_s1(x, w, k, H, W, cin, cout, prev=None):
    n = x.shape[0]
    OH = H + 3 - k
    P = _rup8(W + 2)
    R = H + 3
    M = OH * P
    concat = cin <= 128
    wk = w.reshape(k * k * cin, cout) if concat else w.reshape(k * k, cin, cout)
    if prev is None:
        in_specs = [pl.BlockSpec((1, H, W, cin), lambda i: (i, 0, 0, 0)),
                    _full_spec(wk.shape)]
        args = [x, wk]
        Pin = None
        inv_cnt = None
    else:
        stats, g, b, Pin = prev
        inv_cnt = 1.0 / float(n * H * W)
        in_specs = [pl.BlockSpec((1, H * Pin, cin), lambda i: (i, 0, 0)),
                    _full_spec((2, cin)),
                    _full_spec((1, cin)),
                    _full_spec((1, cin)),
                    _full_spec(wk.shape)]
        args = [x, stats, g.reshape(1, cin), b.reshape(1, cin), wk]
    outs = (jax.ShapeDtypeStruct((n, M, cout), _F32),
            jax.ShapeDtypeStruct((n, 2, cout), _F32))
    out_specs = (pl.BlockSpec((1, M, cout), lambda i: (i, 0, 0)),
                 pl.BlockSpec((1, 2, cout), lambda i: (i, 0, 0)))
    raw, st = _call(_make_s1_body(k, H, W, cin, P, R, M, concat, Pin, inv_cnt),
                    n, in_specs, args, outs, out_specs,
                    [pltpu.VMEM((R, P, cin), _BF16)])
    return raw, jnp.sum(st, axis=0), P


def _make_s2_body(H, W, cin, P0, R, oh, ow, concat, xla_planes):
    Hh, Ph = R // 2, P0 // 2
    M = oh * Ph

    def body(*refs):
        x_ref, w_ref, wd_ref, o_ref, so_ref, od_ref, sd_ref = refs[:7]
        if xla_planes:
            planes = [[x_ref[0, 2 * a + b].reshape(Hh * Ph, cin)
                       for b in (0, 1)] for a in (0, 1)]
        else:
            scr = refs[7]
            _zero_pad_to_scratch(scr, x_ref[0], H, W, P0, R)
            planes = []
            for a in (0, 1):
                planes.append(
                    [scr[pl.ds(a, Hh, 2), pl.ds(b, Ph, 2), :][:, :, 0:cin]
                     .astype(_BF16).reshape(Hh * Ph, cin) for b in (0, 1)])
        if concat:
            av = jnp.concatenate(
                [planes[di % 2][dj % 2]
                 [(di // 2) * Ph + (dj // 2):(di // 2) * Ph + (dj // 2) + M, :]
                 for di in range(3) for dj in range(3)], axis=1)
            acc = jnp.dot(av, w_ref[...], preferred_element_type=_F32)
        else:
            acc = None
            for di in range(3):
                for dj in range(3):
                    off = (di // 2) * Ph + (dj // 2)
                    d = jnp.dot(planes[di % 2][dj % 2][off:off + M, :],
                                w_ref[di * 3 + dj],
                                preferred_element_type=_F32)
                    acc = d if acc is None else acc + d
        _store_stats(so_ref, acc, Ph, ow)
        o_ref[0] = acc
        accd = jnp.dot(planes[1][1][0:M, :], wd_ref[...],
                       preferred_element_type=_F32)
        _store_stats(sd_ref, accd)
        od_ref[0] = accd

    return body


def _xla_planes(y, Hh, Ph):
    n, H, W, c = y.shape
    xp = jnp.pad(y, ((0, 0), (1, 1), (1, 1), (0, 0)))
    ps = []
    for a in (0, 1):
        for b in (0, 1):
            p = xp[:, a::2, b::2, :]
            p = jnp.pad(p, ((0, 0), (0, Hh - p.shape[1]),
                            (0, Ph - p.shape[2]), (0, 0)))
            ps.append(p)
    return jnp.stack(ps, axis=1)


def _conv_s2_down(x, w, wd, H, W, cin, cout):
    n = x.shape[0]
    oh, ow = H // 2, W // 2
    P0 = _rup8(W + 2)
    while (P0 // 2) % 8 != 0:
        P0 += 8
    R = H + 4
    Hh, Ph = R // 2, P0 // 2
    M = oh * Ph
    concat = cin <= 128
    use_xla_planes = cin > 128
    wk = w.reshape(9 * cin, cout) if concat else w.reshape(9, cin, cout)
    wdk = wd.reshape(cin, cout)
    if use_xla_planes:
        xin = _xla_planes(x, Hh, Ph)
        in_specs = [pl.BlockSpec((1, 4, Hh, Ph, cin),
                                 lambda i: (i, 0, 0, 0, 0)),
                    _full_spec(wk.shape), _full_spec(wdk.shape)]
        args = [xin, wk, wdk]
        scratch = []
    else:
        in_specs = [pl.BlockSpec((1, H, W, cin), lambda i: (i, 0, 0, 0)),
                    _full_spec(wk.shape), _full_spec(wdk.shape)]
        args = [x, wk, wdk]
        scratch = [pltpu.VMEM((R, P0, 128), _F32)]
    outs = (jax.ShapeDtypeStruct((n, M, cout), _F32),
            jax.ShapeDtypeStruct((n, 2, cout), _F32),
            jax.ShapeDtypeStruct((n, M, cout), _F32),
            jax.ShapeDtypeStruct((n, 2, cout), _F32))
    out_specs = (pl.BlockSpec((1, M, cout), lambda i: (i, 0, 0)),
                 pl.BlockSpec((1, 2, cout), lambda i: (i, 0, 0)),
                 pl.BlockSpec((1, M, cout), lambda i: (i, 0, 0)),
                 pl.BlockSpec((1, 2, cout), lambda i: (i, 0, 0)))
    raw, st, rawd, std = _call(
        _make_s2_body(H, W, cin, P0, R, oh, ow, concat, use_xla_planes),
        n, in_specs, args, outs, out_specs, scratch)
    return raw, jnp.sum(st, axis=0), rawd, jnp.sum(std, axis=0), Ph


def _make_bnc_body(OH, OW, c, P, Pd, inv_cnt):
    down = Pd is not None

    def body(*refs):
        if down:
            (x_ref, s_ref, g_ref, b_ref, xd_ref, sd_ref, gd_ref, bd_ref,
             o_ref) = refs
            scaled, shiftd = _bn_coeffs(sd_ref[...], gd_ref[...], bd_ref[...],
                                        inv_cnt)
            rd = xd_ref[0].reshape(OH, Pd, c)[:, :OW, :]
            ident = rd.astype(_F32) * scaled[None] + shiftd[None]
        else:
            x_ref, s_ref, g_ref, b_ref, r_ref, o_ref = refs
            ident = r_ref[0].astype(_F32)
        scale, shift = _bn_coeffs(s_ref[...], g_ref[...], b_ref[...], inv_cnt)
        raw = x_ref[0].reshape(OH, P, c)[:, :OW, :]
        y = raw.astype(_F32) * scale[None] + shift[None] + ident
        o_ref[0] = jnp.maximum(y, 0.0).astype(_BF16)

    return body


def _bn_residual(raw, st, g, b, res, OH, OW, c, P):
    n = raw.shape[0]
    inv_cnt = 1.0 / float(n * OH * OW)
    base = [pl.BlockSpec((1, OH * P, c), lambda i: (i, 0, 0)),
            _full_spec((2, c)), _full_spec((1, c)), _full_spec((1, c))]
    args = [raw, st, g.reshape(1, c), b.reshape(1, c)]
    if isinstance(res, tuple):
        rawd, std, gd, bd, Pd = res
        in_specs = base + [pl.BlockSpec((1, OH * Pd, c), lambda i: (i, 0, 0)),
                           _full_spec((2, c)), _full_spec((1, c)),
                           _full_spec((1, c))]
        args += [rawd, std, gd.reshape(1, c), bd.reshape(1, c)]
    else:
        Pd = None
        in_specs = base + [pl.BlockSpec((1, OH, OW, c), lambda i: (i, 0, 0, 0))]
        args += [res]
    outs = jax.ShapeDtypeStruct((n, OH, OW, c), _BF16)
    out_specs = pl.BlockSpec((1, OH, OW, c), lambda i: (i, 0, 0, 0))
    return _call(_make_bnc_body(OH, OW, c, P, Pd, inv_cnt),
                 n, in_specs, args, outs, out_specs)


def _make_stem_pool_body(OH, OW, c, P, inv_cnt):
    R = 114
    Hh, Ph = R // 2, P // 2
    po = OW // 2 + 1 if OW % 2 else OW // 2

    def body(x_ref, s_ref, g_ref, b_ref, o_ref, scr):
        scale, shift = _bn_coeffs(s_ref[...], g_ref[...], b_ref[...], inv_cnt)
        raw = x_ref[0].reshape(OH, P, c)[:, :OW, :]
        y = jnp.maximum(raw.astype(_F32) * scale[None] + shift[None], 0.0)
        _zero_pad_to_scratch(scr, y, OH, OW, P, R)
        planes = []
        for a in (0, 1):
            planes.append([scr[pl.ds(a, Hh, 2), pl.ds(b, Ph, 2), :]
                           for b in (0, 1)])
        r = None
        for a, i0 in ((0, 0), (0, 1), (1, 0)):
            for b2, j0 in ((0, 0), (0, 1), (1, 0)):
                s = planes[a][b2][i0:i0 + po, j0:j0 + po, :]
                r = s if r is None else jnp.maximum(r, s)
        o_ref[0] = r[:, :, 0:c].astype(_BF16)

    return body


def _stem_pool(raw, st, g, b, OH, OW, c, P):
    n = raw.shape[0]
    inv_cnt = 1.0 / float(n * OH * OW)
    po = 56
    in_specs = [pl.BlockSpec((1, OH * P, c), lambda i: (i, 0, 0)),
                _full_spec((2, c)), _full_spec((1, c)), _full_spec((1, c))]
    args = [raw, st, g.reshape(1, c), b.reshape(1, c)]
    outs = jax.ShapeDtypeStruct((n, po, po, c), _BF16)
    out_specs = pl.BlockSpec((1, po, po, c), lambda i: (i, 0, 0, 0))
    return _call(_make_stem_pool_body(OH, OW, c, P, inv_cnt),
                 n, in_specs, args, outs, out_specs,
                 [pltpu.VMEM((114, P, 128), _F32)])


def _head_body(x_ref, w_ref, b_ref, o_ref):
    m = jnp.mean(x_ref[...].astype(_F32), axis=1).astype(_BF16)
    o_ref[...] = jnp.dot(m, w_ref[...],
                         preferred_element_type=_F32) + b_ref[...]


def _head(y, fc_w, fc_b):
    n, h, w, c = y.shape
    x3 = y.reshape(n, h * w, c)
    ncls = fc_w.shape[1]
    return pl.pallas_call(
        _head_body,
        out_shape=jax.ShapeDtypeStruct((n, ncls), _F32),
        grid=(1,),
        in_specs=[_full_spec((n, h * w, c)), _full_spec((c, ncls)),
                  _full_spec((1, ncls))],
        out_specs=pl.BlockSpec((n, ncls), lambda i: (0, 0)),
        compiler_params=pltpu.CompilerParams(
            dimension_semantics=("arbitrary",),
            vmem_limit_bytes=_VMEM_LIMIT),
    )(x3, fc_w, fc_b.reshape(1, ncls).astype(_F32))


def _stem_s2d(x_nchw):
    n = x_nchw.shape[0]
    x = jnp.transpose(x_nchw, (0, 2, 3, 1)).astype(_BF16)
    x = x.reshape(n, 112, 2, 112, 2, 3)
    x = jnp.transpose(x, (0, 1, 3, 2, 4, 5)).reshape(n, 112, 112, 12)
    return jnp.pad(x, ((0, 0), (0, 0), (0, 0), (0, 4)))


def _stem_w(conv1_w):
    w = conv1_w.reshape(7, 7, 3, 64)
    w = jnp.pad(w, ((0, 1), (0, 1), (0, 0), (0, 0)))
    w = w.reshape(4, 2, 4, 2, 3, 64).transpose(0, 2, 1, 3, 4, 5)
    w = w.reshape(16, 12, 64)
    w = jnp.pad(w, ((0, 0), (0, 4), (0, 0)))
    return w.reshape(16 * 16, 64)


def _block_s1(y, w1, g1, b1, w2, g2, b2, H, c):
    raw1, st1, P1 = _conv_s1(y, w1, 3, H, H, c, c)
    raw2, st2, P2 = _conv_s1(raw1, w2, 3, H, H, c, c,
                             prev=(st1, g1, b1, P1))
    return _bn_residual(raw2, st2, g2, b2, y, H, H, c, P2)


def _block_down(y, w1, g1, b1, w2, g2, b2, wd, gd, bd, H, cin, cout):
    oh = H // 2
    raw1, st1, rawd, std, Ph = _conv_s2_down(y, w1, wd, H, H, cin, cout)
    raw2, st2, P2 = _conv_s1(raw1, w2, 3, oh, oh, cout, cout,
                             prev=(st1, g1, b1, Ph))
    return _bn_residual(raw2, st2, g2, b2, (rawd, std, gd, bd, Ph),
                        oh, oh, cout, P2)


def kernel(x, conv1_w, bn1_g, bn1_b,
           L1_b0_conv1, L1_b0_bn1_g, L1_b0_bn1_b, L1_b0_conv2, L1_b0_bn2_g,
           L1_b0_bn2_b, L1_b1_conv1, L1_b1_bn1_g, L1_b1_bn1_b, L1_b1_conv2,
           L1_b1_bn2_g, L1_b1_bn2_b,
           L2_b0_conv1, L2_b0_bn1_g, L2_b0_bn1_b, L2_b0_conv2, L2_b0_bn2_g,
           L2_b0_bn2_b, L2_b0_down_conv, L2_b0_down_bn_g, L2_b0_down_bn_b,
           L2_b1_conv1, L2_b1_bn1_g, L2_b1_bn1_b, L2_b1_conv2, L2_b1_bn2_g,
           L2_b1_bn2_b,
           L3_b0_conv1, L3_b0_bn1_g, L3_b0_bn1_b, L3_b0_conv2, L3_b0_bn2_g,
           L3_b0_bn2_b, L3_b0_down_conv, L3_b0_down_bn_g, L3_b0_down_bn_b,
           L3_b1_conv1, L3_b1_bn1_g, L3_b1_bn1_b, L3_b1_conv2, L3_b1_bn2_g,
           L3_b1_bn2_b,
           L4_b0_conv1, L4_b0_bn1_g, L4_b0_bn1_b, L4_b0_conv2, L4_b0_bn2_g,
           L4_b0_bn2_b, L4_b0_down_conv, L4_b0_down_bn_g, L4_b0_down_bn_b,
           L4_b1_conv1, L4_b1_bn1_g, L4_b1_bn1_b, L4_b1_conv2, L4_b1_bn2_g,
           L4_b1_bn2_b,
           fc_w, fc_b):
    xs = _stem_s2d(x)
    raw0, st0, P0 = _conv_s1(xs, _stem_w(conv1_w), 4, 112, 112, 16, 64)
    y = _stem_pool(raw0, st0, bn1_g, bn1_b, 111, 111, 64, P0)

    y = _block_s1(y, L1_b0_conv1, L1_b0_bn1_g, L1_b0_bn1_b,
                  L1_b0_conv2, L1_b0_bn2_g, L1_b0_bn2_b, 56, 64)
    y = _block_s1(y, L1_b1_conv1, L1_b1_bn1_g, L1_b1_bn1_b,
                  L1_b1_conv2, L1_b1_bn2_g, L1_b1_bn2_b, 56, 64)

    y = _block_down(y, L2_b0_conv1, L2_b0_bn1_g, L2_b0_bn1_b,
                    L2_b0_conv2, L2_b0_bn2_g, L2_b0_bn2_b,
                    L2_b0_down_conv, L2_b0_down_bn_g, L2_b0_down_bn_b,
                    56, 64, 128)
    y = _block_s1(y, L2_b1_conv1, L2_b1_bn1_g, L2_b1_bn1_b,
                  L2_b1_conv2, L2_b1_bn2_g, L2_b1_bn2_b, 28, 128)

    y = _block_down(y, L3_b0_conv1, L3_b0_bn1_g, L3_b0_bn1_b,
                    L3_b0_conv2, L3_b0_bn2_g, L3_b0_bn2_b,
                    L3_b0_down_conv, L3_b0_down_bn_g, L3_b0_down_bn_b,
                    28, 128, 256)
    y = _block_s1(y, L3_b1_conv1, L3_b1_bn1_g, L3_b1_bn1_b,
                  L3_b1_conv2, L3_b1_bn2_g, L3_b1_bn2_b, 14, 256)

    y = _block_down(y, L4_b0_conv1, L4_b0_bn1_g, L4_b0_bn1_b,
                    L4_b0_conv2, L4_b0_bn2_g, L4_b0_bn2_b,
                    L4_b0_down_conv, L4_b0_down_bn_g, L4_b0_down_bn_b,
                    14, 256, 512)
    y = _block_s1(y, L4_b1_conv1, L4_b1_bn1_g, L4_b1_bn1_b,
                  L4_b1_conv2, L4_b1_bn2_g, L4_b1_bn2_b, 7, 512)

    return _head(y, fc_w, fc_b)
